# baseline (speedup 1.0000x reference)
import functools

import jax
import jax.numpy as jnp
from jax import lax
from jax.experimental import pallas as pl
from jax.experimental.pallas import tpu as pltpu
from jax.experimental.pallas import tpu_sc as plsc

E = 8
TOPK = 2
D = 1024
FF = 512
NS = 2
NSFF = NS * FF
ROUTE_SCALE = 1.0
T = 2048
P = T * TOPK
BLK = 128
NSLOT = P + E * BLK
TRASH = NSLOT
SWLEN = NSLOT + 8
NBLK = NSLOT // BLK
NBLKP = 48
NC = 2
NSC = 16
NW = NC * NSC
CPW = P // NW
VPW = CPW // 16
NVR = P // 16
GCH = 32
HALF = D // NC


def _gate_body(x_ref, gw_ref, gb_ref, eid_ref, wts_ref):
    xb = x_ref[...]
    scores = lax.dot_general(xb, gw_ref[...], (((1,), (1,)), ((), ())),
                             preferred_element_type=jnp.float32)
    scores = jax.nn.softmax(scores, axis=-1)
    biased = scores + gb_ref[...]
    lanes = lax.broadcasted_iota(jnp.int32, (T, E), 1)
    i1 = jnp.argmax(biased, axis=-1)[:, None]
    w1 = jnp.sum(jnp.where(lanes == i1, scores, 0.0), axis=-1, keepdims=True)
    masked = jnp.where(lanes == i1, -jnp.inf, biased)
    i2 = jnp.argmax(masked, axis=-1)[:, None]
    w2 = jnp.sum(jnp.where(lanes == i2, scores, 0.0), axis=-1, keepdims=True)
    eid_ref[...] = jnp.concatenate([i1, i2], axis=1)
    wts_ref[...] = jnp.concatenate([w1, w2], axis=1) * ROUTE_SCALE


@jax.jit
def _gate(x2, gate_w, gate_b2):
    return pl.pallas_call(
        _gate_body,
        out_shape=(jax.ShapeDtypeStruct((T, TOPK), jnp.int32),
                   jax.ShapeDtypeStruct((T, TOPK), jnp.float32)),
    )(x2, gate_w, gate_b2)


def _shared_body(xb16_ref, ws1_ref, bs1_ref, ws3_ref, bs3_ref,
                 ws2_ref, bs2_ref, z_ref):
    xb = xb16_ref[...]
    h1 = lax.dot_general(xb, ws1_ref[...], (((1,), (1,)), ((), ())),
                         preferred_element_type=jnp.float32) + bs1_ref[...]
    h3 = lax.dot_general(xb, ws3_ref[...], (((1,), (1,)), ((), ())),
                         preferred_element_type=jnp.float32) + bs3_ref[...]
    h = ((h1 * jax.nn.sigmoid(h1)) * h3).astype(jnp.bfloat16)
    z_ref[...] = lax.dot_general(h, ws2_ref[...], (((1,), (1,)), ((), ())),
                                 preferred_element_type=jnp.float32) + bs2_ref[...]


@jax.jit
def _shared(xb16, Ws1b, bs1r, Ws3b, bs3r, Ws2b, bs2r):
    TBS = 1024
    return pl.pallas_call(
        _shared_body,
        grid=(T // TBS,),
        in_specs=[
            pl.BlockSpec((TBS, D), lambda t: (t, 0)),
            pl.BlockSpec((NSFF, D), lambda t: (0, 0)),
            pl.BlockSpec((1, NSFF), lambda t: (0, 0)),
            pl.BlockSpec((NSFF, D), lambda t: (0, 0)),
            pl.BlockSpec((1, NSFF), lambda t: (0, 0)),
            pl.BlockSpec((D, NSFF), lambda t: (0, 0)),
            pl.BlockSpec((1, D), lambda t: (0, 0)),
        ],
        out_specs=pl.BlockSpec((TBS, D), lambda t: (t, 0)),
        out_shape=jax.ShapeDtypeStruct((T, D), jnp.float32),
    )(xb16, Ws1b, bs1r, Ws3b, bs3r, Ws2b, bs2r)


def _dispatch_body(eflat, wflat, x2,
                   xs, slot_w, pair_slot, block_expert,
                   eids_v, wbuf, toks_v, slots_v,
                   gidx, sidx, gbuf0, gbuf1, gbuf2,
                   zbuf, zidx, bev,
                   semw, semt, semz,
                   gsem0, gsem1, gsem2, ssem0, ssem1, ssem2):
    c = lax.axis_index("c")
    s = lax.axis_index("s")
    wid = s * NC + c

    pltpu.sync_copy(eflat, eids_v)
    pltpu.sync_copy(wflat.at[pl.ds(wid * CPW, CPW)], wbuf)

    zeros16 = jnp.zeros((16,), jnp.int32)
    lane16 = lax.broadcasted_iota(jnp.int32, (16,), 0)

    for j in range(VPW):
        toks = lax.shift_right_logical(
            lane16 + wid * CPW + j * 16, 1).astype(jnp.int32)
        toks_v[pl.ds(j * 16, 16)] = toks
        gidx[j // 2, pl.ds((j % 2) * 16, 16)] = toks

    nch = CPW // GCH
    bufs = [gbuf0, gbuf1, gbuf2]
    gsems = [gsem0, gsem1, gsem2]
    ssems = [ssem0, ssem1, ssem2]
    gcopies = [None] * nch
    scopies = [None] * nch
    for ci in range(min(3, nch)):
        gcopies[ci] = pltpu.make_async_copy(
            x2.at[gidx.at[ci]], bufs[ci % 3], gsems[ci % 3])
        gcopies[ci].start()

    def count_step(j, carry):
        tot, pre = carry
        v = eids_v[pl.ds(j * 16, 16)]
        in_prefix = j < wid * VPW
        new_tot = []
        new_pre = []
        for e in range(E):
            ind = jnp.where(v == e, 1, 0)
            new_tot.append(tot[e] + ind)
            new_pre.append(pre[e] + jnp.where(in_prefix, ind, zeros16))
        return tuple(new_tot), tuple(new_pre)

    tot_v, pre_v = lax.fori_loop(
        0, NVR, count_step,
        (tuple(zeros16 for _ in range(E)), tuple(zeros16 for _ in range(E))))
    tot = [jnp.sum(tot_v[e]) for e in range(E)]
    pre = [jnp.sum(pre_v[e]) for e in range(E)]
    cap = [((tot[e] + (BLK - 1)) // BLK) * BLK for e in range(E)]
    base = [jnp.int32(0)]
    for e in range(E):
        base.append(base[e] + cap[e])

    carry = [jnp.int32(0)] * E
    for j in range(VPW):
        v = eids_v[pl.ds(wid * CPW + j * 16, 16)]
        slots = zeros16
        for e in range(E):
            ind = jnp.where(v == e, 1, 0)
            r = plsc.cumsum(ind)
            slots = slots + ind * (base[e] + pre[e] + carry[e] - 1 + r)
            carry[e] = carry[e] + r[15]
        slots_v[pl.ds(j * 16, 16)] = slots
        sidx[j // 2, pl.ds((j % 2) * 16, 16)] = slots

    cw = pltpu.make_async_copy(wbuf, slot_w.at[slots_v], semw)
    cw.start()
    ct = pltpu.make_async_copy(sidx, pair_slot.at[wid], semt)
    ct.start()

    for ci in range(nch):
        b = ci % 3
        gcopies[ci].wait()
        if ci >= 3:
            pass
        scopies[ci] = pltpu.make_async_copy(
            bufs[b], xs.at[sidx.at[ci]], ssems[b])
        scopies[ci].start()
        if ci + 3 < nch:
            scopies[ci].wait()
            gcopies[ci + 3] = pltpu.make_async_copy(
                x2.at[gidx.at[ci + 3]], bufs[b], gsems[b])
            gcopies[ci + 3].start()
    for ci in range(max(0, nch - 3), nch):
        scopies[ci].wait()

    @pl.when(wid < 16)
    def _zero_pad():
        for j in range(VPW):
            zbuf[pl.ds(j * 16, 16)] = jnp.zeros((16,), jnp.float32)
        for j in range(VPW):
            i = lane16 + j * 16
            in_expert = wid < E
            ew = jnp.where(in_expert, wid, 0)
            start_e = jnp.int32(0)
            tot_e = jnp.int32(0)
            for e in range(E):
                sel = jnp.where(ew == e, 1, 0)
                start_e = start_e + sel * (base[e] + tot[e])
                tot_e = tot_e + sel * (cap[e] - tot[e])
            start_t = base[E] + (wid - E) * CPW
            start = jnp.where(in_expert, start_e, start_t)
            limit = jnp.where(in_expert, start_e + tot_e, jnp.int32(NSLOT))
            tgt = start + i
            tgt = jnp.where(tgt < limit, tgt, jnp.int32(TRASH))
            zidx[pl.ds(j * 16, 16)] = tgt
        cz = pltpu.make_async_copy(zbuf, slot_w.at[zidx], semz)
        cz.start()
        cz.wait()

    @pl.when(wid == 0)
    def _block_expert():
        for t in range(NBLKP // 16):
            bid = lane16 + t * 16
            acc = jnp.zeros((16,), jnp.int32)
            for e in range(E):
                lo = base[e] // BLK
                hi = (base[e] + cap[e]) // BLK
                acc = acc + e * jnp.where((bid >= lo) & (bid < hi), 1, 0)
            bev[pl.ds(t * 16, 16)] = acc
        pltpu.sync_copy(bev, block_expert)

    cw.wait()
    ct.wait()


@jax.jit
def _dispatch(eflat, wflat, x2):
    mesh = plsc.VectorSubcoreMesh(core_axis_name="c", subcore_axis_name="s",
                                  num_cores=NC, num_subcores=NSC)
    f = pl.kernel(
        _dispatch_body,
        out_type=(jax.ShapeDtypeStruct((SWLEN, D), jnp.float32),
                  jax.ShapeDtypeStruct((SWLEN,), jnp.float32),
                  jax.ShapeDtypeStruct((NW, CPW // GCH, GCH), jnp.int32),
                  jax.ShapeDtypeStruct((NBLKP,), jnp.int32)),
        mesh=mesh,
        compiler_params=pltpu.CompilerParams(needs_layout_passes=False),
        scratch_types=[
            pltpu.VMEM((P,), jnp.int32),
            pltpu.VMEM((CPW,), jnp.float32),
            pltpu.VMEM((CPW,), jnp.int32),
            pltpu.VMEM((CPW,), jnp.int32),
            pltpu.VMEM((CPW // GCH, GCH), jnp.int32),
            pltpu.VMEM((CPW // GCH, GCH), jnp.int32),
            pltpu.VMEM((GCH, D), jnp.float32),
            pltpu.VMEM((GCH, D), jnp.float32),
            pltpu.VMEM((GCH, D), jnp.float32),
            pltpu.VMEM((CPW,), jnp.float32),
            pltpu.VMEM((CPW,), jnp.int32),
            pltpu.VMEM((NBLKP,), jnp.int32),
            pltpu.SemaphoreType.DMA,
            pltpu.SemaphoreType.DMA,
            pltpu.SemaphoreType.DMA,
            pltpu.SemaphoreType.DMA,
            pltpu.SemaphoreType.DMA,
            pltpu.SemaphoreType.DMA,
            pltpu.SemaphoreType.DMA,
            pltpu.SemaphoreType.DMA,
            pltpu.SemaphoreType.DMA,
        ],
    )
    return f(eflat, wflat, x2)


def _gffn_body(be_ref, xs_ref, wcol_ref, w1_ref, b1_ref, w3_ref, b3_ref,
               w2_ref, b2_ref, ys_ref):
    xb = xs_ref[...].astype(jnp.bfloat16)
    h1 = lax.dot_general(xb, w1_ref[0], (((1,), (1,)), ((), ())),
                         preferred_element_type=jnp.float32) + b1_ref[0]
    h3 = lax.dot_general(xb, w3_ref[0], (((1,), (1,)), ((), ())),
                         preferred_element_type=jnp.float32) + b3_ref[0]
    h = ((h1 * jax.nn.sigmoid(h1)) * h3).astype(jnp.bfloat16)
    ye = lax.dot_general(h, w2_ref[0], (((1,), (1,)), ((), ())),
                         preferred_element_type=jnp.float32) + b2_ref[0]
    ye = jnp.where(jnp.abs(ye) < jnp.inf, ye, 0.0)
    ys_ref[...] = ye * wcol_ref[...]


@jax.jit
def _gffn(block_expert, xs, wcol, W1b, b1r, W3b, b3r, W2b, b2r):
    grid_spec = pltpu.PrefetchScalarGridSpec(
        num_scalar_prefetch=1,
        grid=(NBLK,),
        in_specs=[
            pl.BlockSpec((BLK, D), lambda b, be: (b, 0)),
            pl.BlockSpec((BLK, 1), lambda b, be: (b, 0)),
            pl.BlockSpec((1, FF, D), lambda b, be: (be[b], 0, 0)),
            pl.BlockSpec((1, 1, FF), lambda b, be: (be[b], 0, 0)),
            pl.BlockSpec((1, FF, D), lambda b, be: (be[b], 0, 0)),
            pl.BlockSpec((1, 1, FF), lambda b, be: (be[b], 0, 0)),
            pl.BlockSpec((1, D, FF), lambda b, be: (be[b], 0, 0)),
            pl.BlockSpec((1, 1, D), lambda b, be: (be[b], 0, 0)),
        ],
        out_specs=pl.BlockSpec((BLK, D), lambda b, be: (b, 0)),
    )
    return pl.pallas_call(
        _gffn_body,
        grid_spec=grid_spec,
        out_shape=jax.ShapeDtypeStruct((NSLOT, D), jnp.float32),
    )(block_expert, xs, wcol, W1b, b1r, W3b, b3r, W2b, b2r)


TPW = T // NW


def _combine_body(ys, pair_slot, z, y2,
                  psl, ybuf0, ybuf1, zbuf, outbuf0, outbuf1,
                  gsem0, gsem1, zsem, osem0, osem1):
    c = lax.axis_index("c")
    s = lax.axis_index("s")
    wid = s * NC + c

    pltpu.sync_copy(pair_slot.at[wid], psl)

    nch = CPW // GCH
    bufs = [ybuf0, ybuf1]
    gsems = [gsem0, gsem1]
    gcopies = [None] * nch
    gcopies[0] = pltpu.make_async_copy(ys.at[psl.at[0]], bufs[0], gsems[0])
    gcopies[0].start()
    ocopies = [None] * nch
    for ci in range(nch):
        b = ci % 2
        if ci + 1 < nch:
            gcopies[ci + 1] = pltpu.make_async_copy(
                ys.at[psl.at[ci + 1]], bufs[1 - b], gsems[1 - b])
            gcopies[ci + 1].start()
        t0 = wid * TPW + ci * (GCH // 2)
        zc = pltpu.make_async_copy(z.at[pl.ds(t0, GCH // 2)], zbuf, zsem)
        zc.start()
        gcopies[ci].wait()
        zc.wait()
        if ci >= 2:
            ocopies[ci - 2].wait()
        ybuf = bufs[b]
        outbuf = [outbuf0, outbuf1][b]

        def addrow(j, _):
            for l in range(D // 16):
                sl = pl.ds(l * 16, 16)
                outbuf[j, sl] = ybuf[2 * j, sl] + ybuf[2 * j + 1, sl] + zbuf[j, sl]
            return 0
        lax.fori_loop(0, GCH // 2, addrow, 0)
        ocopies[ci] = pltpu.make_async_copy(
            outbuf, y2.at[pl.ds(t0, GCH // 2)], [osem0, osem1][b])
        ocopies[ci].start()
    ocopies[nch - 2].wait()
    ocopies[nch - 1].wait()


@jax.jit
def _combine(ys, pair_slot, z):
    mesh = plsc.VectorSubcoreMesh(core_axis_name="c", subcore_axis_name="s",
                                  num_cores=NC, num_subcores=NSC)
    f = pl.kernel(
        _combine_body,
        out_type=jax.ShapeDtypeStruct((T, D), jnp.float32),
        mesh=mesh,
        compiler_params=pltpu.CompilerParams(needs_layout_passes=False),
        scratch_types=[
            pltpu.VMEM((CPW // GCH, GCH), jnp.int32),
            pltpu.VMEM((GCH, D), jnp.float32),
            pltpu.VMEM((GCH, D), jnp.float32),
            pltpu.VMEM((GCH // 2, D), jnp.float32),
            pltpu.VMEM((GCH // 2, D), jnp.float32),
            pltpu.VMEM((GCH // 2, D), jnp.float32),
            pltpu.SemaphoreType.DMA,
            pltpu.SemaphoreType.DMA,
            pltpu.SemaphoreType.DMA,
            pltpu.SemaphoreType.DMA,
            pltpu.SemaphoreType.DMA,
        ],
    )
    return f(ys, pair_slot, z)


def kernel(x, gate_w, gate_b, W1, b1, W2, b2, W3, b3,
           Ws1, bs1, Ws2, bs2, Ws3, bs3):
    shape = x.shape
    x2 = x.reshape(T, D)
    xb16 = x2.astype(jnp.bfloat16)

    eids, wts = _gate(x2, gate_w, gate_b.reshape(1, E))
    z = _shared(xb16, Ws1.astype(jnp.bfloat16), bs1.reshape(1, NSFF),
                Ws3.astype(jnp.bfloat16), bs3.reshape(1, NSFF),
                Ws2.astype(jnp.bfloat16), bs2.reshape(1, D))
    xs, slot_w, pair_slot, block_expert = _dispatch(
        eids.reshape(P), wts.reshape(P), x2)
    ys = _gffn(block_expert, xs, slot_w[:NSLOT].reshape(NSLOT, 1),
               W1.astype(jnp.bfloat16), b1.reshape(E, 1, FF),
               W3.astype(jnp.bfloat16), b3.reshape(E, 1, FF),
               W2.astype(jnp.bfloat16), b2.reshape(E, 1, D))
    y2 = _combine(ys, pair_slot, z)
    return y2.reshape(shape)


# speedup vs baseline: 1.0040x; 1.0040x over previous
import functools

import jax
import jax.numpy as jnp
from jax import lax
from jax.experimental import pallas as pl
from jax.experimental.pallas import tpu as pltpu
from jax.experimental.pallas import tpu_sc as plsc

E = 8
TOPK = 2
D = 1024
FF = 512
NS = 2
NSFF = NS * FF
ROUTE_SCALE = 1.0
T = 2048
P = T * TOPK
BLK = 128
NSLOT = P + E * BLK
TRASH = NSLOT
SWLEN = NSLOT + 8
NBLK = NSLOT // BLK
NBLKP = 48
NC = 2
NSC = 16
NW = NC * NSC
CPW = P // NW
VPW = CPW // 16
NVR = P // 16
GCH = 32
HALF = D // NC


def _gate_body(x_ref, gw_ref, gb_ref, eid_ref, wts_ref, cnt_ref):
    xb = x_ref[...]
    scores = lax.dot_general(xb, gw_ref[...], (((1,), (1,)), ((), ())),
                             preferred_element_type=jnp.float32)
    scores = jax.nn.softmax(scores, axis=-1)
    biased = scores + gb_ref[...]
    lanes = lax.broadcasted_iota(jnp.int32, (T, E), 1)
    i1 = jnp.argmax(biased, axis=-1)[:, None]
    w1 = jnp.sum(jnp.where(lanes == i1, scores, 0.0), axis=-1, keepdims=True)
    masked = jnp.where(lanes == i1, -jnp.inf, biased)
    i2 = jnp.argmax(masked, axis=-1)[:, None]
    w2 = jnp.sum(jnp.where(lanes == i2, scores, 0.0), axis=-1, keepdims=True)
    eid_ref[...] = jnp.concatenate([i1, i2], axis=1)
    wts_ref[...] = jnp.concatenate([w1, w2], axis=1) * ROUTE_SCALE
    oh = (jnp.where(lanes == i1, 1.0, 0.0)
          + jnp.where(lanes == i2, 1.0, 0.0))
    rw = lax.broadcasted_iota(jnp.int32, (NW, T), 0)
    ct = lax.broadcasted_iota(jnp.int32, (NW, T), 1)
    sel = jnp.where(rw == ct // (T // NW), 1.0, 0.0)
    cnt = lax.dot_general(oh, sel, (((0,), (1,)), ((), ())),
                          preferred_element_type=jnp.float32)
    cnt_ref[...] = cnt.astype(jnp.int32)


@jax.jit
def _gate(x2, gate_w, gate_b2):
    return pl.pallas_call(
        _gate_body,
        out_shape=(jax.ShapeDtypeStruct((T, TOPK), jnp.int32),
                   jax.ShapeDtypeStruct((T, TOPK), jnp.float32),
                   jax.ShapeDtypeStruct((E, NW), jnp.int32)),
    )(x2, gate_w, gate_b2)


def _shared_body(xb16_ref, ws1_ref, bs1_ref, ws3_ref, bs3_ref,
                 ws2_ref, bs2_ref, z_ref):
    xb = xb16_ref[...]
    h1 = lax.dot_general(xb, ws1_ref[...], (((1,), (1,)), ((), ())),
                         preferred_element_type=jnp.float32) + bs1_ref[...]
    h3 = lax.dot_general(xb, ws3_ref[...], (((1,), (1,)), ((), ())),
                         preferred_element_type=jnp.float32) + bs3_ref[...]
    h = ((h1 * jax.nn.sigmoid(h1)) * h3).astype(jnp.bfloat16)
    z_ref[...] = lax.dot_general(h, ws2_ref[...], (((1,), (1,)), ((), ())),
                                 preferred_element_type=jnp.float32) + bs2_ref[...]


@jax.jit
def _shared(xb16, Ws1b, bs1r, Ws3b, bs3r, Ws2b, bs2r):
    TBS = 1024
    return pl.pallas_call(
        _shared_body,
        grid=(T // TBS,),
        in_specs=[
            pl.BlockSpec((TBS, D), lambda t: (t, 0)),
            pl.BlockSpec((NSFF, D), lambda t: (0, 0)),
            pl.BlockSpec((1, NSFF), lambda t: (0, 0)),
            pl.BlockSpec((NSFF, D), lambda t: (0, 0)),
            pl.BlockSpec((1, NSFF), lambda t: (0, 0)),
            pl.BlockSpec((D, NSFF), lambda t: (0, 0)),
            pl.BlockSpec((1, D), lambda t: (0, 0)),
        ],
        out_specs=pl.BlockSpec((TBS, D), lambda t: (t, 0)),
        out_shape=jax.ShapeDtypeStruct((T, D), jnp.float32),
    )(xb16, Ws1b, bs1r, Ws3b, bs3r, Ws2b, bs2r)


def _dispatch_body(eflat, wflat, cnt8, x2,
                   xs, slot_w, pair_slot, block_expert,
                   eids_v, wbuf, cntv, slots_v,
                   gidx, sidx, gbuf0, gbuf1, gbuf2,
                   zbuf, zidx, bev,
                   semw, semt, semz,
                   gsem0, gsem1, gsem2, ssem0, ssem1, ssem2):
    c = lax.axis_index("c")
    s = lax.axis_index("s")
    wid = s * NC + c

    pltpu.sync_copy(eflat.at[pl.ds(wid * CPW, CPW)], eids_v)
    pltpu.sync_copy(wflat.at[pl.ds(wid * CPW, CPW)], wbuf)
    pltpu.sync_copy(cnt8, cntv)

    zeros16 = jnp.zeros((16,), jnp.int32)
    lane16 = lax.broadcasted_iota(jnp.int32, (16,), 0)

    for j in range(VPW):
        toks = lax.shift_right_logical(
            lane16 + wid * CPW + j * 16, 1).astype(jnp.int32)
        gidx[j // 2, pl.ds((j % 2) * 16, 16)] = toks

    nch = CPW // GCH
    bufs = [gbuf0, gbuf1, gbuf2]
    gsems = [gsem0, gsem1, gsem2]
    ssems = [ssem0, ssem1, ssem2]
    gcopies = [None] * nch
    scopies = [None] * nch
    for ci in range(min(3, nch)):
        gcopies[ci] = pltpu.make_async_copy(
            x2.at[gidx.at[ci]], bufs[ci % 3], gsems[ci % 3])
        gcopies[ci].start()

    tot = []
    pre = []
    for e in range(E):
        v0 = cntv[e, pl.ds(0, 16)]
        v1 = cntv[e, pl.ds(16, 16)]
        tot.append(jnp.sum(v0) + jnp.sum(v1))
        pre.append(jnp.sum(jnp.where(lane16 < wid, v0, zeros16))
                   + jnp.sum(jnp.where(lane16 + 16 < wid, v1, zeros16)))
    cap = [((tot[e] + (BLK - 1)) // BLK) * BLK for e in range(E)]
    base = [jnp.int32(0)]
    for e in range(E):
        base.append(base[e] + cap[e])

    carry = [jnp.int32(0)] * E
    for j in range(VPW):
        v = eids_v[pl.ds(j * 16, 16)]
        slots = zeros16
        for e in range(E):
            ind = jnp.where(v == e, 1, 0)
            r = plsc.cumsum(ind)
            slots = slots + ind * (base[e] + pre[e] + carry[e] - 1 + r)
            carry[e] = carry[e] + r[15]
        slots_v[pl.ds(j * 16, 16)] = slots
        sidx[j // 2, pl.ds((j % 2) * 16, 16)] = slots

    cw = pltpu.make_async_copy(wbuf, slot_w.at[slots_v], semw)
    cw.start()
    ct = pltpu.make_async_copy(sidx, pair_slot.at[wid], semt)
    ct.start()

    for ci in range(nch):
        b = ci % 3
        gcopies[ci].wait()
        if ci >= 3:
            pass
        scopies[ci] = pltpu.make_async_copy(
            bufs[b], xs.at[sidx.at[ci]], ssems[b])
        scopies[ci].start()
        if ci + 3 < nch:
            scopies[ci].wait()
            gcopies[ci + 3] = pltpu.make_async_copy(
                x2.at[gidx.at[ci + 3]], bufs[b], gsems[b])
            gcopies[ci + 3].start()
    for ci in range(max(0, nch - 3), nch):
        scopies[ci].wait()

    @pl.when(wid < 16)
    def _zero_pad():
        for j in range(VPW):
            zbuf[pl.ds(j * 16, 16)] = jnp.zeros((16,), jnp.float32)
        for j in range(VPW):
            i = lane16 + j * 16
            in_expert = wid < E
            ew = jnp.where(in_expert, wid, 0)
            start_e = jnp.int32(0)
            tot_e = jnp.int32(0)
            for e in range(E):
                sel = jnp.where(ew == e, 1, 0)
                start_e = start_e + sel * (base[e] + tot[e])
                tot_e = tot_e + sel * (cap[e] - tot[e])
            start_t = base[E] + (wid - E) * CPW
            start = jnp.where(in_expert, start_e, start_t)
            limit = jnp.where(in_expert, start_e + tot_e, jnp.int32(NSLOT))
            tgt = start + i
            tgt = jnp.where(tgt < limit, tgt, jnp.int32(TRASH))
            zidx[pl.ds(j * 16, 16)] = tgt
        cz = pltpu.make_async_copy(zbuf, slot_w.at[zidx], semz)
        cz.start()
        cz.wait()

    @pl.when(wid == 0)
    def _block_expert():
        for t in range(NBLKP // 16):
            bid = lane16 + t * 16
            acc = jnp.zeros((16,), jnp.int32)
            for e in range(E):
                lo = base[e] // BLK
                hi = (base[e] + cap[e]) // BLK
                acc = acc + e * jnp.where((bid >= lo) & (bid < hi), 1, 0)
            bev[pl.ds(t * 16, 16)] = acc
        pltpu.sync_copy(bev, block_expert)

    cw.wait()
    ct.wait()


@jax.jit
def _dispatch(eflat, wflat, cnt8, x2):
    mesh = plsc.VectorSubcoreMesh(core_axis_name="c", subcore_axis_name="s",
                                  num_cores=NC, num_subcores=NSC)
    f = pl.kernel(
        _dispatch_body,
        out_type=(jax.ShapeDtypeStruct((SWLEN, D), jnp.float32),
                  jax.ShapeDtypeStruct((SWLEN,), jnp.float32),
                  jax.ShapeDtypeStruct((NW, CPW // GCH, GCH), jnp.int32),
                  jax.ShapeDtypeStruct((NBLKP,), jnp.int32)),
        mesh=mesh,
        compiler_params=pltpu.CompilerParams(needs_layout_passes=False),
        scratch_types=[
            pltpu.VMEM((CPW,), jnp.int32),
            pltpu.VMEM((CPW,), jnp.float32),
            pltpu.VMEM((E, NW), jnp.int32),
            pltpu.VMEM((CPW,), jnp.int32),
            pltpu.VMEM((CPW // GCH, GCH), jnp.int32),
            pltpu.VMEM((CPW // GCH, GCH), jnp.int32),
            pltpu.VMEM((GCH, D), jnp.float32),
            pltpu.VMEM((GCH, D), jnp.float32),
            pltpu.VMEM((GCH, D), jnp.float32),
            pltpu.VMEM((CPW,), jnp.float32),
            pltpu.VMEM((CPW,), jnp.int32),
            pltpu.VMEM((NBLKP,), jnp.int32),
            pltpu.SemaphoreType.DMA,
            pltpu.SemaphoreType.DMA,
            pltpu.SemaphoreType.DMA,
            pltpu.SemaphoreType.DMA,
            pltpu.SemaphoreType.DMA,
            pltpu.SemaphoreType.DMA,
            pltpu.SemaphoreType.DMA,
            pltpu.SemaphoreType.DMA,
            pltpu.SemaphoreType.DMA,
        ],
    )
    return f(eflat, wflat, cnt8, x2)


def _gffn_body(be_ref, xs_ref, wcol_ref, w1_ref, b1_ref, w3_ref, b3_ref,
               w2_ref, b2_ref, ys_ref):
    xb = xs_ref[...].astype(jnp.bfloat16)
    h1 = lax.dot_general(xb, w1_ref[0], (((1,), (1,)), ((), ())),
                         preferred_element_type=jnp.float32) + b1_ref[0]
    h3 = lax.dot_general(xb, w3_ref[0], (((1,), (1,)), ((), ())),
                         preferred_element_type=jnp.float32) + b3_ref[0]
    h = ((h1 * jax.nn.sigmoid(h1)) * h3).astype(jnp.bfloat16)
    ye = lax.dot_general(h, w2_ref[0], (((1,), (1,)), ((), ())),
                         preferred_element_type=jnp.float32) + b2_ref[0]
    ye = jnp.where(jnp.abs(ye) < jnp.inf, ye, 0.0)
    ys_ref[...] = ye * wcol_ref[...]


@jax.jit
def _gffn(block_expert, xs, wcol, W1b, b1r, W3b, b3r, W2b, b2r):
    grid_spec = pltpu.PrefetchScalarGridSpec(
        num_scalar_prefetch=1,
        grid=(NBLK,),
        in_specs=[
            pl.BlockSpec((BLK, D), lambda b, be: (b, 0)),
            pl.BlockSpec((BLK, 1), lambda b, be: (b, 0)),
            pl.BlockSpec((1, FF, D), lambda b, be: (be[b], 0, 0)),
            pl.BlockSpec((1, 1, FF), lambda b, be: (be[b], 0, 0)),
            pl.BlockSpec((1, FF, D), lambda b, be: (be[b], 0, 0)),
            pl.BlockSpec((1, 1, FF), lambda b, be: (be[b], 0, 0)),
            pl.BlockSpec((1, D, FF), lambda b, be: (be[b], 0, 0)),
            pl.BlockSpec((1, 1, D), lambda b, be: (be[b], 0, 0)),
        ],
        out_specs=pl.BlockSpec((BLK, D), lambda b, be: (b, 0)),
    )
    return pl.pallas_call(
        _gffn_body,
        grid_spec=grid_spec,
        out_shape=jax.ShapeDtypeStruct((NSLOT, D), jnp.float32),
    )(block_expert, xs, wcol, W1b, b1r, W3b, b3r, W2b, b2r)


TPW = T // NW


def _combine_body(ys, pair_slot, z, y2,
                  psl, ybuf0, ybuf1, zbuf, outbuf0, outbuf1,
                  gsem0, gsem1, zsem, osem0, osem1):
    c = lax.axis_index("c")
    s = lax.axis_index("s")
    wid = s * NC + c

    pltpu.sync_copy(pair_slot.at[wid], psl)

    nch = CPW // GCH
    bufs = [ybuf0, ybuf1]
    gsems = [gsem0, gsem1]
    gcopies = [None] * nch
    gcopies[0] = pltpu.make_async_copy(ys.at[psl.at[0]], bufs[0], gsems[0])
    gcopies[0].start()
    ocopies = [None] * nch
    for ci in range(nch):
        b = ci % 2
        if ci + 1 < nch:
            gcopies[ci + 1] = pltpu.make_async_copy(
                ys.at[psl.at[ci + 1]], bufs[1 - b], gsems[1 - b])
            gcopies[ci + 1].start()
        t0 = wid * TPW + ci * (GCH // 2)
        zc = pltpu.make_async_copy(z.at[pl.ds(t0, GCH // 2)], zbuf, zsem)
        zc.start()
        gcopies[ci].wait()
        zc.wait()
        if ci >= 2:
            ocopies[ci - 2].wait()
        ybuf = bufs[b]
        outbuf = [outbuf0, outbuf1][b]

        def addrow(j, _):
            for l in range(D // 16):
                sl = pl.ds(l * 16, 16)
                outbuf[j, sl] = ybuf[2 * j, sl] + ybuf[2 * j + 1, sl] + zbuf[j, sl]
            return 0
        lax.fori_loop(0, GCH // 2, addrow, 0)
        ocopies[ci] = pltpu.make_async_copy(
            outbuf, y2.at[pl.ds(t0, GCH // 2)], [osem0, osem1][b])
        ocopies[ci].start()
    ocopies[nch - 2].wait()
    ocopies[nch - 1].wait()


@jax.jit
def _combine(ys, pair_slot, z):
    mesh = plsc.VectorSubcoreMesh(core_axis_name="c", subcore_axis_name="s",
                                  num_cores=NC, num_subcores=NSC)
    f = pl.kernel(
        _combine_body,
        out_type=jax.ShapeDtypeStruct((T, D), jnp.float32),
        mesh=mesh,
        compiler_params=pltpu.CompilerParams(needs_layout_passes=False),
        scratch_types=[
            pltpu.VMEM((CPW // GCH, GCH), jnp.int32),
            pltpu.VMEM((GCH, D), jnp.float32),
            pltpu.VMEM((GCH, D), jnp.float32),
            pltpu.VMEM((GCH // 2, D), jnp.float32),
            pltpu.VMEM((GCH // 2, D), jnp.float32),
            pltpu.VMEM((GCH // 2, D), jnp.float32),
            pltpu.SemaphoreType.DMA,
            pltpu.SemaphoreType.DMA,
            pltpu.SemaphoreType.DMA,
            pltpu.SemaphoreType.DMA,
            pltpu.SemaphoreType.DMA,
        ],
    )
    return f(ys, pair_slot, z)


def kernel(x, gate_w, gate_b, W1, b1, W2, b2, W3, b3,
           Ws1, bs1, Ws2, bs2, Ws3, bs3):
    shape = x.shape
    x2 = x.reshape(T, D)
    xb16 = x2.astype(jnp.bfloat16)

    eids, wts, cnt8 = _gate(x2, gate_w, gate_b.reshape(1, E))
    z = _shared(xb16, Ws1.astype(jnp.bfloat16), bs1.reshape(1, NSFF),
                Ws3.astype(jnp.bfloat16), bs3.reshape(1, NSFF),
                Ws2.astype(jnp.bfloat16), bs2.reshape(1, D))
    xs, slot_w, pair_slot, block_expert = _dispatch(
        eids.reshape(P), wts.reshape(P), cnt8, x2)
    ys = _gffn(block_expert, xs, slot_w[:NSLOT].reshape(NSLOT, 1),
               W1.astype(jnp.bfloat16), b1.reshape(E, 1, FF),
               W3.astype(jnp.bfloat16), b3.reshape(E, 1, FF),
               W2.astype(jnp.bfloat16), b2.reshape(E, 1, D))
    y2 = _combine(ys, pair_slot, z)
    return y2.reshape(shape)


# speedup vs baseline: 1.6859x; 1.6791x over previous
import functools

import jax
import jax.numpy as jnp
from jax import lax
from jax.experimental import pallas as pl
from jax.experimental.pallas import tpu as pltpu
from jax.experimental.pallas import tpu_sc as plsc

E = 8
TOPK = 2
D = 1024
FF = 512
NS = 2
NSFF = NS * FF
ROUTE_SCALE = 1.0
T = 2048
P = T * TOPK
BLK = 128
NSLOT = P + E * BLK
TRASH = NSLOT
SWLEN = NSLOT + 8
NBLK = NSLOT // BLK
NBLKP = 48
NC = 2
NSC = 16
NW = NC * NSC
CPW = P // NW
VPW = CPW // 16
NVR = P // 16
GCH = 32
HALF = D // NC


def _gate_body(x_ref, gw_ref, gb_ref, eid_ref, wts_ref, cnt_ref):
    xb = x_ref[...]
    scores = lax.dot_general(xb, gw_ref[...], (((1,), (1,)), ((), ())),
                             preferred_element_type=jnp.float32)
    scores = jax.nn.softmax(scores, axis=-1)
    biased = scores + gb_ref[...]
    lanes = lax.broadcasted_iota(jnp.int32, (T, E), 1)
    i1 = jnp.argmax(biased, axis=-1)[:, None]
    w1 = jnp.sum(jnp.where(lanes == i1, scores, 0.0), axis=-1, keepdims=True)
    masked = jnp.where(lanes == i1, -jnp.inf, biased)
    i2 = jnp.argmax(masked, axis=-1)[:, None]
    w2 = jnp.sum(jnp.where(lanes == i2, scores, 0.0), axis=-1, keepdims=True)
    eid_ref[...] = jnp.concatenate([i1, i2], axis=1)
    wts_ref[...] = jnp.concatenate([w1, w2], axis=1) * ROUTE_SCALE
    oh = (jnp.where(lanes == i1, 1.0, 0.0)
          + jnp.where(lanes == i2, 1.0, 0.0))
    rw = lax.broadcasted_iota(jnp.int32, (NW, T), 0)
    ct = lax.broadcasted_iota(jnp.int32, (NW, T), 1)
    sel = jnp.where(rw == ct // (T // NW), 1.0, 0.0)
    cnt = lax.dot_general(oh, sel, (((0,), (1,)), ((), ())),
                          preferred_element_type=jnp.float32)
    cnt_ref[...] = cnt.astype(jnp.int32)


@jax.jit
def _gate(x2, gate_w, gate_b2):
    return pl.pallas_call(
        _gate_body,
        out_shape=(jax.ShapeDtypeStruct((T, TOPK), jnp.int32),
                   jax.ShapeDtypeStruct((T, TOPK), jnp.float32),
                   jax.ShapeDtypeStruct((E, NW), jnp.int32)),
    )(x2, gate_w, gate_b2)


def _shared_body(xb16_ref, ws1_ref, bs1_ref, ws3_ref, bs3_ref,
                 ws2_ref, bs2_ref, z_ref):
    xb = xb16_ref[...]
    h1 = lax.dot_general(xb, ws1_ref[...], (((1,), (1,)), ((), ())),
                         preferred_element_type=jnp.float32) + bs1_ref[...]
    h3 = lax.dot_general(xb, ws3_ref[...], (((1,), (1,)), ((), ())),
                         preferred_element_type=jnp.float32) + bs3_ref[...]
    h = ((h1 * jax.nn.sigmoid(h1)) * h3).astype(jnp.bfloat16)
    z_ref[...] = lax.dot_general(h, ws2_ref[...], (((1,), (1,)), ((), ())),
                                 preferred_element_type=jnp.float32) + bs2_ref[...]


@jax.jit
def _shared(xb16, Ws1b, bs1r, Ws3b, bs3r, Ws2b, bs2r):
    TBS = 1024
    return pl.pallas_call(
        _shared_body,
        grid=(T // TBS,),
        in_specs=[
            pl.BlockSpec((TBS, D), lambda t: (t, 0)),
            pl.BlockSpec((NSFF, D), lambda t: (0, 0)),
            pl.BlockSpec((1, NSFF), lambda t: (0, 0)),
            pl.BlockSpec((NSFF, D), lambda t: (0, 0)),
            pl.BlockSpec((1, NSFF), lambda t: (0, 0)),
            pl.BlockSpec((D, NSFF), lambda t: (0, 0)),
            pl.BlockSpec((1, D), lambda t: (0, 0)),
        ],
        out_specs=pl.BlockSpec((TBS, D), lambda t: (t, 0)),
        out_shape=jax.ShapeDtypeStruct((T, D), jnp.float32),
    )(xb16, Ws1b, bs1r, Ws3b, bs3r, Ws2b, bs2r)


def _dispatch_body(eflat, cnt8, x2,
                   xs, pair_slot, block_expert,
                   eids_v, cntv, slots_v,
                   gidx, sidx, gbuf0, gbuf1, gbuf2, bev,
                   semt,
                   gsem0, gsem1, gsem2, ssem0, ssem1, ssem2):
    c = lax.axis_index("c")
    s = lax.axis_index("s")
    wid = s * NC + c

    pltpu.sync_copy(eflat.at[pl.ds(wid * CPW, CPW)], eids_v)
    pltpu.sync_copy(cnt8, cntv)

    zeros16 = jnp.zeros((16,), jnp.int32)
    lane16 = lax.broadcasted_iota(jnp.int32, (16,), 0)

    for j in range(VPW):
        toks = lax.shift_right_logical(
            lane16 + wid * CPW + j * 16, 1).astype(jnp.int32)
        gidx[j // 2, pl.ds((j % 2) * 16, 16)] = toks

    nch = CPW // GCH
    bufs = [gbuf0, gbuf1, gbuf2]
    gsems = [gsem0, gsem1, gsem2]
    ssems = [ssem0, ssem1, ssem2]
    gcopies = [None] * nch
    scopies = [None] * nch
    for ci in range(min(3, nch)):
        gcopies[ci] = pltpu.make_async_copy(
            x2.at[gidx.at[ci]], bufs[ci % 3], gsems[ci % 3])
        gcopies[ci].start()

    tot = []
    pre = []
    for e in range(E):
        v0 = cntv[e, pl.ds(0, 16)]
        v1 = cntv[e, pl.ds(16, 16)]
        tot.append(jnp.sum(v0) + jnp.sum(v1))
        pre.append(jnp.sum(jnp.where(lane16 < wid, v0, zeros16))
                   + jnp.sum(jnp.where(lane16 + 16 < wid, v1, zeros16)))
    cap = [((tot[e] + (BLK - 1)) // BLK) * BLK for e in range(E)]
    base = [jnp.int32(0)]
    for e in range(E):
        base.append(base[e] + cap[e])

    carry = [jnp.int32(0)] * E
    for j in range(VPW):
        v = eids_v[pl.ds(j * 16, 16)]
        slots = zeros16
        for e in range(E):
            ind = jnp.where(v == e, 1, 0)
            r = plsc.cumsum(ind)
            slots = slots + ind * (base[e] + pre[e] + carry[e] - 1 + r)
            carry[e] = carry[e] + r[15]
        slots_v[pl.ds(j * 16, 16)] = slots
        sidx[j // 2, pl.ds((j % 2) * 16, 16)] = slots

    ct = pltpu.make_async_copy(sidx, pair_slot.at[wid], semt)
    ct.start()

    for ci in range(nch):
        b = ci % 3
        gcopies[ci].wait()
        if ci >= 3:
            pass
        scopies[ci] = pltpu.make_async_copy(
            bufs[b], xs.at[sidx.at[ci]], ssems[b])
        scopies[ci].start()
        if ci + 3 < nch:
            scopies[ci].wait()
            gcopies[ci + 3] = pltpu.make_async_copy(
                x2.at[gidx.at[ci + 3]], bufs[b], gsems[b])
            gcopies[ci + 3].start()
    for ci in range(max(0, nch - 3), nch):
        scopies[ci].wait()

    @pl.when(wid == 0)
    def _block_expert():
        for t in range(NBLKP // 16):
            bid = lane16 + t * 16
            acc = jnp.zeros((16,), jnp.int32)
            for e in range(E):
                lo = base[e] // BLK
                hi = (base[e] + cap[e]) // BLK
                acc = acc + e * jnp.where((bid >= lo) & (bid < hi), 1, 0)
            bev[pl.ds(t * 16, 16)] = acc
        pltpu.sync_copy(bev, block_expert)

    ct.wait()


@jax.jit
def _dispatch(eflat, cnt8, x2):
    mesh = plsc.VectorSubcoreMesh(core_axis_name="c", subcore_axis_name="s",
                                  num_cores=NC, num_subcores=NSC)
    f = pl.kernel(
        _dispatch_body,
        out_type=(jax.ShapeDtypeStruct((SWLEN, D), jnp.float32),
                  jax.ShapeDtypeStruct((NW, CPW // GCH, GCH), jnp.int32),
                  jax.ShapeDtypeStruct((NBLKP,), jnp.int32)),
        mesh=mesh,
        compiler_params=pltpu.CompilerParams(needs_layout_passes=False),
        scratch_types=[
            pltpu.VMEM((CPW,), jnp.int32),
            pltpu.VMEM((E, NW), jnp.int32),
            pltpu.VMEM((CPW,), jnp.int32),
            pltpu.VMEM((CPW // GCH, GCH), jnp.int32),
            pltpu.VMEM((CPW // GCH, GCH), jnp.int32),
            pltpu.VMEM((GCH, D), jnp.float32),
            pltpu.VMEM((GCH, D), jnp.float32),
            pltpu.VMEM((GCH, D), jnp.float32),
            pltpu.VMEM((NBLKP,), jnp.int32),
            pltpu.SemaphoreType.DMA,
            pltpu.SemaphoreType.DMA,
            pltpu.SemaphoreType.DMA,
            pltpu.SemaphoreType.DMA,
            pltpu.SemaphoreType.DMA,
            pltpu.SemaphoreType.DMA,
            pltpu.SemaphoreType.DMA,
        ],
    )
    return f(eflat, cnt8, x2)


def _gffn_body(be_ref, xs_ref, w1_ref, b1_ref, w3_ref, b3_ref,
               w2_ref, b2_ref, ys_ref):
    xb = xs_ref[...].astype(jnp.bfloat16)
    h1 = lax.dot_general(xb, w1_ref[0], (((1,), (1,)), ((), ())),
                         preferred_element_type=jnp.float32) + b1_ref[0]
    h3 = lax.dot_general(xb, w3_ref[0], (((1,), (1,)), ((), ())),
                         preferred_element_type=jnp.float32) + b3_ref[0]
    h = ((h1 * jax.nn.sigmoid(h1)) * h3).astype(jnp.bfloat16)
    ye = lax.dot_general(h, w2_ref[0], (((1,), (1,)), ((), ())),
                         preferred_element_type=jnp.float32) + b2_ref[0]
    ys_ref[...] = jnp.where(jnp.abs(ye) < jnp.inf, ye, 0.0)


@jax.jit
def _gffn(block_expert, xs, W1b, b1r, W3b, b3r, W2b, b2r):
    grid_spec = pltpu.PrefetchScalarGridSpec(
        num_scalar_prefetch=1,
        grid=(NBLK,),
        in_specs=[
            pl.BlockSpec((BLK, D), lambda b, be: (b, 0)),
            pl.BlockSpec((1, FF, D), lambda b, be: (be[b], 0, 0)),
            pl.BlockSpec((1, 1, FF), lambda b, be: (be[b], 0, 0)),
            pl.BlockSpec((1, FF, D), lambda b, be: (be[b], 0, 0)),
            pl.BlockSpec((1, 1, FF), lambda b, be: (be[b], 0, 0)),
            pl.BlockSpec((1, D, FF), lambda b, be: (be[b], 0, 0)),
            pl.BlockSpec((1, 1, D), lambda b, be: (be[b], 0, 0)),
        ],
        out_specs=pl.BlockSpec((BLK, D), lambda b, be: (b, 0)),
    )
    return pl.pallas_call(
        _gffn_body,
        grid_spec=grid_spec,
        out_shape=jax.ShapeDtypeStruct((NSLOT, D), jnp.float32),
    )(block_expert, xs, W1b, b1r, W3b, b3r, W2b, b2r)


TPW = T // NW


def _combine_body(ys, pair_slot, wflat, z, y2,
                  psl, wv, ybuf0, ybuf1, zbuf, outbuf0, outbuf1,
                  gsem0, gsem1, zsem, osem0, osem1):
    c = lax.axis_index("c")
    s = lax.axis_index("s")
    wid = s * NC + c

    pltpu.sync_copy(pair_slot.at[wid], psl)
    pltpu.sync_copy(wflat.at[pl.ds(wid * CPW, CPW)], wv)

    nch = CPW // GCH
    bufs = [ybuf0, ybuf1]
    gsems = [gsem0, gsem1]
    gcopies = [None] * nch
    gcopies[0] = pltpu.make_async_copy(ys.at[psl.at[0]], bufs[0], gsems[0])
    gcopies[0].start()
    ocopies = [None] * nch
    for ci in range(nch):
        b = ci % 2
        if ci + 1 < nch:
            gcopies[ci + 1] = pltpu.make_async_copy(
                ys.at[psl.at[ci + 1]], bufs[1 - b], gsems[1 - b])
            gcopies[ci + 1].start()
        t0 = wid * TPW + ci * (GCH // 2)
        zc = pltpu.make_async_copy(z.at[pl.ds(t0, GCH // 2)], zbuf, zsem)
        zc.start()
        gcopies[ci].wait()
        zc.wait()
        if ci >= 2:
            ocopies[ci - 2].wait()
        ybuf = bufs[b]
        outbuf = [outbuf0, outbuf1][b]

        def addrow(j, _):
            pe = ci * GCH + 2 * j
            we = plsc.load_gather(wv, [jnp.full((16,), pe, jnp.int32)])
            wo = plsc.load_gather(wv, [jnp.full((16,), pe + 1, jnp.int32)])
            for l in range(D // 16):
                sl = pl.ds(l * 16, 16)
                outbuf[j, sl] = (we * ybuf[2 * j, sl]
                                 + wo * ybuf[2 * j + 1, sl] + zbuf[j, sl])
            return 0
        lax.fori_loop(0, GCH // 2, addrow, 0)
        ocopies[ci] = pltpu.make_async_copy(
            outbuf, y2.at[pl.ds(t0, GCH // 2)], [osem0, osem1][b])
        ocopies[ci].start()
    ocopies[nch - 2].wait()
    ocopies[nch - 1].wait()


@jax.jit
def _combine(ys, pair_slot, wflat, z):
    mesh = plsc.VectorSubcoreMesh(core_axis_name="c", subcore_axis_name="s",
                                  num_cores=NC, num_subcores=NSC)
    f = pl.kernel(
        _combine_body,
        out_type=jax.ShapeDtypeStruct((T, D), jnp.float32),
        mesh=mesh,
        compiler_params=pltpu.CompilerParams(needs_layout_passes=False),
        scratch_types=[
            pltpu.VMEM((CPW // GCH, GCH), jnp.int32),
            pltpu.VMEM((CPW,), jnp.float32),
            pltpu.VMEM((GCH, D), jnp.float32),
            pltpu.VMEM((GCH, D), jnp.float32),
            pltpu.VMEM((GCH // 2, D), jnp.float32),
            pltpu.VMEM((GCH // 2, D), jnp.float32),
            pltpu.VMEM((GCH // 2, D), jnp.float32),
            pltpu.SemaphoreType.DMA,
            pltpu.SemaphoreType.DMA,
            pltpu.SemaphoreType.DMA,
            pltpu.SemaphoreType.DMA,
            pltpu.SemaphoreType.DMA,
        ],
    )
    return f(ys, pair_slot, wflat, z)


def kernel(x, gate_w, gate_b, W1, b1, W2, b2, W3, b3,
           Ws1, bs1, Ws2, bs2, Ws3, bs3):
    shape = x.shape
    x2 = x.reshape(T, D)
    xb16 = x2.astype(jnp.bfloat16)

    eids, wts, cnt8 = _gate(x2, gate_w, gate_b.reshape(1, E))
    z = _shared(xb16, Ws1.astype(jnp.bfloat16), bs1.reshape(1, NSFF),
                Ws3.astype(jnp.bfloat16), bs3.reshape(1, NSFF),
                Ws2.astype(jnp.bfloat16), bs2.reshape(1, D))
    xs, pair_slot, block_expert = _dispatch(eids.reshape(P), cnt8, x2)
    ys = _gffn(block_expert, xs,
               W1.astype(jnp.bfloat16), b1.reshape(E, 1, FF),
               W3.astype(jnp.bfloat16), b3.reshape(E, 1, FF),
               W2.astype(jnp.bfloat16), b2.reshape(E, 1, D))
    y2 = _combine(ys, pair_slot, wts.reshape(P), z)
    return y2.reshape(shape)


# speedup vs baseline: 1.7330x; 1.0279x over previous
import functools

import jax
import jax.numpy as jnp
from jax import lax
from jax.experimental import pallas as pl
from jax.experimental.pallas import tpu as pltpu
from jax.experimental.pallas import tpu_sc as plsc

E = 8
TOPK = 2
D = 1024
FF = 512
NS = 2
NSFF = NS * FF
ROUTE_SCALE = 1.0
T = 2048
P = T * TOPK
BLK = 128
NSLOT = P + E * BLK
TRASH = NSLOT
SWLEN = NSLOT + 8
NBLK = NSLOT // BLK
NBLKP = 48
NC = 2
NSC = 16
NW = NC * NSC
CPW = P // NW
VPW = CPW // 16
NVR = P // 16
GCH = 32
HALF = D // NC


def _shared_body(x_ref, xb16_ref, gw_ref, gb_ref, ws1_ref, bs1_ref,
                 ws3_ref, bs3_ref, ws2_ref, bs2_ref,
                 z_ref, eid_ref, wts_ref, cnt_ref):
    TBS = z_ref.shape[0]
    xb32 = x_ref[...]
    scores = lax.dot_general(xb32, gw_ref[...], (((1,), (1,)), ((), ())),
                             preferred_element_type=jnp.float32)
    scores = jax.nn.softmax(scores, axis=-1)
    biased = scores + gb_ref[...]
    lanes = lax.broadcasted_iota(jnp.int32, (TBS, E), 1)
    i1 = jnp.argmax(biased, axis=-1)[:, None]
    w1 = jnp.sum(jnp.where(lanes == i1, scores, 0.0), axis=-1, keepdims=True)
    masked = jnp.where(lanes == i1, -jnp.inf, biased)
    i2 = jnp.argmax(masked, axis=-1)[:, None]
    w2 = jnp.sum(jnp.where(lanes == i2, scores, 0.0), axis=-1, keepdims=True)
    eid_ref[...] = jnp.concatenate([i1, i2], axis=1)
    wts_ref[...] = jnp.concatenate([w1, w2], axis=1) * ROUTE_SCALE
    npw = TBS // (T // NW)
    oh = (jnp.where(lanes == i1, 1.0, 0.0)
          + jnp.where(lanes == i2, 1.0, 0.0))
    rw = lax.broadcasted_iota(jnp.int32, (npw, TBS), 0)
    ctc = lax.broadcasted_iota(jnp.int32, (npw, TBS), 1)
    sel = jnp.where(rw == ctc // (T // NW), 1.0, 0.0)
    cnt = lax.dot_general(oh, sel, (((0,), (1,)), ((), ())),
                          preferred_element_type=jnp.float32)
    cnt_ref[...] = cnt.astype(jnp.int32)[None]

    xb = xb16_ref[...]
    h1 = lax.dot_general(xb, ws1_ref[...], (((1,), (1,)), ((), ())),
                         preferred_element_type=jnp.float32) + bs1_ref[...]
    h3 = lax.dot_general(xb, ws3_ref[...], (((1,), (1,)), ((), ())),
                         preferred_element_type=jnp.float32) + bs3_ref[...]
    h = ((h1 * jax.nn.sigmoid(h1)) * h3).astype(jnp.bfloat16)
    z_ref[...] = lax.dot_general(h, ws2_ref[...], (((1,), (1,)), ((), ())),
                                 preferred_element_type=jnp.float32) + bs2_ref[...]


@jax.jit
def _shared(x2, xb16, gate_w, gate_b2, Ws1b, bs1r, Ws3b, bs3r, Ws2b, bs2r):
    TBS = 1024
    npw = TBS // (T // NW)
    return pl.pallas_call(
        _shared_body,
        grid=(T // TBS,),
        in_specs=[
            pl.BlockSpec((TBS, D), lambda t: (t, 0)),
            pl.BlockSpec((TBS, D), lambda t: (t, 0)),
            pl.BlockSpec((E, D), lambda t: (0, 0)),
            pl.BlockSpec((1, E), lambda t: (0, 0)),
            pl.BlockSpec((NSFF, D), lambda t: (0, 0)),
            pl.BlockSpec((1, NSFF), lambda t: (0, 0)),
            pl.BlockSpec((NSFF, D), lambda t: (0, 0)),
            pl.BlockSpec((1, NSFF), lambda t: (0, 0)),
            pl.BlockSpec((D, NSFF), lambda t: (0, 0)),
            pl.BlockSpec((1, D), lambda t: (0, 0)),
        ],
        out_specs=(pl.BlockSpec((TBS, D), lambda t: (t, 0)),
                   pl.BlockSpec((TBS, TOPK), lambda t: (t, 0)),
                   pl.BlockSpec((TBS, TOPK), lambda t: (t, 0)),
                   pl.BlockSpec((1, E, npw), lambda t: (t, 0, 0))),
        out_shape=(jax.ShapeDtypeStruct((T, D), jnp.float32),
                   jax.ShapeDtypeStruct((T, TOPK), jnp.int32),
                   jax.ShapeDtypeStruct((T, TOPK), jnp.float32),
                   jax.ShapeDtypeStruct((T // TBS, E, npw), jnp.int32)),
    )(x2, xb16, gate_w, gate_b2, Ws1b, bs1r, Ws3b, bs3r, Ws2b, bs2r)


def _dispatch_body(eflat, cnt8, x2,
                   xs, pair_slot, block_expert,
                   eids_v, cntv, slots_v,
                   gidx, sidx, gbuf0, gbuf1, gbuf2, bev,
                   semt,
                   gsem0, gsem1, gsem2, ssem0, ssem1, ssem2):
    c = lax.axis_index("c")
    s = lax.axis_index("s")
    wid = s * NC + c

    pltpu.sync_copy(eflat.at[pl.ds(wid * CPW, CPW)], eids_v)
    pltpu.sync_copy(cnt8, cntv)

    zeros16 = jnp.zeros((16,), jnp.int32)
    lane16 = lax.broadcasted_iota(jnp.int32, (16,), 0)

    for j in range(VPW):
        toks = lax.shift_right_logical(
            lane16 + wid * CPW + j * 16, 1).astype(jnp.int32)
        gidx[j // 2, pl.ds((j % 2) * 16, 16)] = toks

    nch = CPW // GCH
    bufs = [gbuf0, gbuf1, gbuf2]
    gsems = [gsem0, gsem1, gsem2]
    ssems = [ssem0, ssem1, ssem2]
    gcopies = [None] * nch
    scopies = [None] * nch
    for ci in range(min(3, nch)):
        gcopies[ci] = pltpu.make_async_copy(
            x2.at[gidx.at[ci]], bufs[ci % 3], gsems[ci % 3])
        gcopies[ci].start()

    tot = []
    pre = []
    for e in range(E):
        v0 = cntv[e, pl.ds(0, 16)]
        v1 = cntv[e, pl.ds(16, 16)]
        tot.append(jnp.sum(v0) + jnp.sum(v1))
        pre.append(jnp.sum(jnp.where(lane16 < wid, v0, zeros16))
                   + jnp.sum(jnp.where(lane16 + 16 < wid, v1, zeros16)))
    cap = [((tot[e] + (BLK - 1)) // BLK) * BLK for e in range(E)]
    base = [jnp.int32(0)]
    for e in range(E):
        base.append(base[e] + cap[e])

    carry = [jnp.int32(0)] * E
    for j in range(VPW):
        v = eids_v[pl.ds(j * 16, 16)]
        slots = zeros16
        for e in range(E):
            ind = jnp.where(v == e, 1, 0)
            r = plsc.cumsum(ind)
            slots = slots + ind * (base[e] + pre[e] + carry[e] - 1 + r)
            carry[e] = carry[e] + r[15]
        slots_v[pl.ds(j * 16, 16)] = slots
        sidx[j // 2, pl.ds((j % 2) * 16, 16)] = slots

    ct = pltpu.make_async_copy(sidx, pair_slot.at[wid], semt)
    ct.start()

    for ci in range(nch):
        b = ci % 3
        gcopies[ci].wait()
        if ci >= 3:
            pass
        scopies[ci] = pltpu.make_async_copy(
            bufs[b], xs.at[sidx.at[ci]], ssems[b])
        scopies[ci].start()
        if ci + 3 < nch:
            scopies[ci].wait()
            gcopies[ci + 3] = pltpu.make_async_copy(
                x2.at[gidx.at[ci + 3]], bufs[b], gsems[b])
            gcopies[ci + 3].start()
    for ci in range(max(0, nch - 3), nch):
        scopies[ci].wait()

    @pl.when(wid == 0)
    def _block_expert():
        for t in range(NBLKP // 16):
            bid = lane16 + t * 16
            acc = jnp.zeros((16,), jnp.int32)
            for e in range(E):
                lo = base[e] // BLK
                hi = (base[e] + cap[e]) // BLK
                acc = acc + e * jnp.where((bid >= lo) & (bid < hi), 1, 0)
            bev[pl.ds(t * 16, 16)] = acc
        pltpu.sync_copy(bev, block_expert)

    ct.wait()


@jax.jit
def _dispatch(eflat, cnt8, x2):
    mesh = plsc.VectorSubcoreMesh(core_axis_name="c", subcore_axis_name="s",
                                  num_cores=NC, num_subcores=NSC)
    f = pl.kernel(
        _dispatch_body,
        out_type=(jax.ShapeDtypeStruct((SWLEN, D), jnp.float32),
                  jax.ShapeDtypeStruct((NW, CPW // GCH, GCH), jnp.int32),
                  jax.ShapeDtypeStruct((NBLKP,), jnp.int32)),
        mesh=mesh,
        compiler_params=pltpu.CompilerParams(needs_layout_passes=False),
        scratch_types=[
            pltpu.VMEM((CPW,), jnp.int32),
            pltpu.VMEM((E, NW), jnp.int32),
            pltpu.VMEM((CPW,), jnp.int32),
            pltpu.VMEM((CPW // GCH, GCH), jnp.int32),
            pltpu.VMEM((CPW // GCH, GCH), jnp.int32),
            pltpu.VMEM((GCH, D), jnp.float32),
            pltpu.VMEM((GCH, D), jnp.float32),
            pltpu.VMEM((GCH, D), jnp.float32),
            pltpu.VMEM((NBLKP,), jnp.int32),
            pltpu.SemaphoreType.DMA,
            pltpu.SemaphoreType.DMA,
            pltpu.SemaphoreType.DMA,
            pltpu.SemaphoreType.DMA,
            pltpu.SemaphoreType.DMA,
            pltpu.SemaphoreType.DMA,
            pltpu.SemaphoreType.DMA,
        ],
    )
    return f(eflat, cnt8, x2)


def _gffn_body(be_ref, xs_ref, w1_ref, b1_ref, w3_ref, b3_ref,
               w2_ref, b2_ref, ys_ref):
    xb = xs_ref[...].astype(jnp.bfloat16)
    h1 = lax.dot_general(xb, w1_ref[0], (((1,), (1,)), ((), ())),
                         preferred_element_type=jnp.float32) + b1_ref[0]
    h3 = lax.dot_general(xb, w3_ref[0], (((1,), (1,)), ((), ())),
                         preferred_element_type=jnp.float32) + b3_ref[0]
    h = ((h1 * jax.nn.sigmoid(h1)) * h3).astype(jnp.bfloat16)
    ye = lax.dot_general(h, w2_ref[0], (((1,), (1,)), ((), ())),
                         preferred_element_type=jnp.float32) + b2_ref[0]
    ys_ref[...] = jnp.where(jnp.abs(ye) < jnp.inf, ye, 0.0)


@jax.jit
def _gffn(block_expert, xs, W1b, b1r, W3b, b3r, W2b, b2r):
    grid_spec = pltpu.PrefetchScalarGridSpec(
        num_scalar_prefetch=1,
        grid=(NBLK,),
        in_specs=[
            pl.BlockSpec((BLK, D), lambda b, be: (b, 0)),
            pl.BlockSpec((1, FF, D), lambda b, be: (be[b], 0, 0)),
            pl.BlockSpec((1, 1, FF), lambda b, be: (be[b], 0, 0)),
            pl.BlockSpec((1, FF, D), lambda b, be: (be[b], 0, 0)),
            pl.BlockSpec((1, 1, FF), lambda b, be: (be[b], 0, 0)),
            pl.BlockSpec((1, D, FF), lambda b, be: (be[b], 0, 0)),
            pl.BlockSpec((1, 1, D), lambda b, be: (be[b], 0, 0)),
        ],
        out_specs=pl.BlockSpec((BLK, D), lambda b, be: (b, 0)),
    )
    return pl.pallas_call(
        _gffn_body,
        grid_spec=grid_spec,
        out_shape=jax.ShapeDtypeStruct((NSLOT, D), jnp.float32),
    )(block_expert, xs, W1b, b1r, W3b, b3r, W2b, b2r)


TPW = T // NW


def _combine_body(ys, pair_slot, wflat, z, y2,
                  psl, wv, ybuf0, ybuf1, zbuf0, zbuf1, outbuf0,
                  gsem0, gsem1, zsem0, zsem1, osem0, osem1):
    c = lax.axis_index("c")
    s = lax.axis_index("s")
    wid = s * NC + c

    pltpu.sync_copy(pair_slot.at[wid], psl)
    pltpu.sync_copy(wflat.at[pl.ds(wid * CPW, CPW)], wv)

    nch = CPW // GCH
    ybufs = [ybuf0, ybuf1]
    zbufs = [zbuf0, zbuf1]
    obufs = [outbuf0, outbuf0]
    gsems = [gsem0, gsem1]
    zsems = [zsem0, zsem1]
    osems = [osem0, osem1]
    t_of = lambda ci: wid * TPW + ci * (GCH // 2)
    gcopies = [None] * nch
    zcopies = [None] * nch
    ocopies = [None] * nch
    gcopies[0] = pltpu.make_async_copy(ys.at[psl.at[0]], ybufs[0], gsems[0])
    gcopies[0].start()
    zcopies[0] = pltpu.make_async_copy(z.at[pl.ds(t_of(0), GCH // 2)],
                                       zbufs[0], zsems[0])
    zcopies[0].start()
    for ci in range(nch):
        b = ci % 2
        if ci + 1 < nch:
            gcopies[ci + 1] = pltpu.make_async_copy(
                ys.at[psl.at[ci + 1]], ybufs[1 - b], gsems[1 - b])
            gcopies[ci + 1].start()
            zcopies[ci + 1] = pltpu.make_async_copy(
                z.at[pl.ds(t_of(ci + 1), GCH // 2)], zbufs[1 - b], zsems[1 - b])
            zcopies[ci + 1].start()
        gcopies[ci].wait()
        zcopies[ci].wait()
        if ci >= 1:
            ocopies[ci - 1].wait()
        ybuf = ybufs[b]
        zbuf = zbufs[b]
        outbuf = obufs[b]

        def addrow(j, _):
            pe = ci * GCH + 2 * j
            we = plsc.load_gather(wv, [jnp.full((16,), pe, jnp.int32)])
            wo = plsc.load_gather(wv, [jnp.full((16,), pe + 1, jnp.int32)])
            for l in range(D // 16):
                sl = pl.ds(l * 16, 16)
                outbuf[j, sl] = (we * ybuf[2 * j, sl]
                                 + wo * ybuf[2 * j + 1, sl] + zbuf[j, sl])
            return 0
        lax.fori_loop(0, GCH // 2, addrow, 0)
        ocopies[ci] = pltpu.make_async_copy(
            outbuf, y2.at[pl.ds(t_of(ci), GCH // 2)], osems[b])
        ocopies[ci].start()
    ocopies[nch - 1].wait()


@jax.jit
def _combine(ys, pair_slot, wflat, z):
    mesh = plsc.VectorSubcoreMesh(core_axis_name="c", subcore_axis_name="s",
                                  num_cores=NC, num_subcores=NSC)
    f = pl.kernel(
        _combine_body,
        out_type=jax.ShapeDtypeStruct((T, D), jnp.float32),
        mesh=mesh,
        compiler_params=pltpu.CompilerParams(needs_layout_passes=False),
        scratch_types=[
            pltpu.VMEM((CPW // GCH, GCH), jnp.int32),
            pltpu.VMEM((CPW,), jnp.float32),
            pltpu.VMEM((GCH, D), jnp.float32),
            pltpu.VMEM((GCH, D), jnp.float32),
            pltpu.VMEM((GCH // 2, D), jnp.float32),
            pltpu.VMEM((GCH // 2, D), jnp.float32),
            pltpu.VMEM((GCH // 2, D), jnp.float32),
            pltpu.SemaphoreType.DMA,
            pltpu.SemaphoreType.DMA,
            pltpu.SemaphoreType.DMA,
            pltpu.SemaphoreType.DMA,
            pltpu.SemaphoreType.DMA,
            pltpu.SemaphoreType.DMA,
        ],
    )
    return f(ys, pair_slot, wflat, z)


def kernel(x, gate_w, gate_b, W1, b1, W2, b2, W3, b3,
           Ws1, bs1, Ws2, bs2, Ws3, bs3):
    shape = x.shape
    x2 = x.reshape(T, D)
    xb16 = x2.astype(jnp.bfloat16)

    z, eids, wts, cnt8 = _shared(
        x2, xb16, gate_w, gate_b.reshape(1, E),
        Ws1.astype(jnp.bfloat16), bs1.reshape(1, NSFF),
        Ws3.astype(jnp.bfloat16), bs3.reshape(1, NSFF),
        Ws2.astype(jnp.bfloat16), bs2.reshape(1, D))
    cnt8f = jnp.concatenate([cnt8[i] for i in range(cnt8.shape[0])], axis=1)
    xs, pair_slot, block_expert = _dispatch(eids.reshape(P), cnt8f, x2)
    ys = _gffn(block_expert, xs,
               W1.astype(jnp.bfloat16), b1.reshape(E, 1, FF),
               W3.astype(jnp.bfloat16), b3.reshape(E, 1, FF),
               W2.astype(jnp.bfloat16), b2.reshape(E, 1, D))
    y2 = _combine(ys, pair_slot, wts.reshape(P), z)
    return y2.reshape(shape)


# speedup vs baseline: 1.9424x; 1.1209x over previous
import functools

import jax
import jax.numpy as jnp
from jax import lax
from jax.experimental import pallas as pl
from jax.experimental.pallas import tpu as pltpu
from jax.experimental.pallas import tpu_sc as plsc

E = 8
TOPK = 2
D = 1024
FF = 512
NS = 2
NSFF = NS * FF
ROUTE_SCALE = 1.0
T = 2048
P = T * TOPK
BLK = 256
NSLOT = P + E * BLK
TRASH = NSLOT
SWLEN = NSLOT + 8
NBLK = NSLOT // BLK
NBLKP = 32
NC = 2
NSC = 16
NW = NC * NSC
CPW = P // NW
VPW = CPW // 16
NVR = P // 16
GCH = 32
HALF = D // NC


def _shared_body(x_ref, xb16_ref, gw_ref, gb_ref, ws1_ref, bs1_ref,
                 ws3_ref, bs3_ref, ws2_ref, bs2_ref,
                 z_ref, eid_ref, wts_ref, cnt_ref):
    TBS = z_ref.shape[0]
    xb32 = x_ref[...]
    scores = lax.dot_general(xb32, gw_ref[...], (((1,), (1,)), ((), ())),
                             preferred_element_type=jnp.float32)
    scores = jax.nn.softmax(scores, axis=-1)
    biased = scores + gb_ref[...]
    lanes = lax.broadcasted_iota(jnp.int32, (TBS, E), 1)
    i1 = jnp.argmax(biased, axis=-1)[:, None]
    w1 = jnp.sum(jnp.where(lanes == i1, scores, 0.0), axis=-1, keepdims=True)
    masked = jnp.where(lanes == i1, -jnp.inf, biased)
    i2 = jnp.argmax(masked, axis=-1)[:, None]
    w2 = jnp.sum(jnp.where(lanes == i2, scores, 0.0), axis=-1, keepdims=True)
    eid_ref[...] = jnp.concatenate([i1, i2], axis=1)
    wts_ref[...] = jnp.concatenate([w1, w2], axis=1) * ROUTE_SCALE
    npw = TBS // (T // NW)
    oh = (jnp.where(lanes == i1, 1.0, 0.0)
          + jnp.where(lanes == i2, 1.0, 0.0))
    rw = lax.broadcasted_iota(jnp.int32, (npw, TBS), 0)
    ctc = lax.broadcasted_iota(jnp.int32, (npw, TBS), 1)
    sel = jnp.where(rw == ctc // (T // NW), 1.0, 0.0)
    cnt = lax.dot_general(oh, sel, (((0,), (1,)), ((), ())),
                          preferred_element_type=jnp.float32)
    cnt_ref[...] = cnt.astype(jnp.int32)[None]

    xb = xb16_ref[...]
    h1 = lax.dot_general(xb, ws1_ref[...], (((1,), (1,)), ((), ())),
                         preferred_element_type=jnp.float32) + bs1_ref[...]
    h3 = lax.dot_general(xb, ws3_ref[...], (((1,), (1,)), ((), ())),
                         preferred_element_type=jnp.float32) + bs3_ref[...]
    h = ((h1 * jax.nn.sigmoid(h1)) * h3).astype(jnp.bfloat16)
    z_ref[...] = lax.dot_general(h, ws2_ref[...], (((1,), (1,)), ((), ())),
                                 preferred_element_type=jnp.float32) + bs2_ref[...]


@jax.jit
def _shared(x2, xb16, gate_w, gate_b2, Ws1b, bs1r, Ws3b, bs3r, Ws2b, bs2r):
    TBS = 1024
    npw = TBS // (T // NW)
    return pl.pallas_call(
        _shared_body,
        grid=(T // TBS,),
        in_specs=[
            pl.BlockSpec((TBS, D), lambda t: (t, 0)),
            pl.BlockSpec((TBS, D), lambda t: (t, 0)),
            pl.BlockSpec((E, D), lambda t: (0, 0)),
            pl.BlockSpec((1, E), lambda t: (0, 0)),
            pl.BlockSpec((NSFF, D), lambda t: (0, 0)),
            pl.BlockSpec((1, NSFF), lambda t: (0, 0)),
            pl.BlockSpec((NSFF, D), lambda t: (0, 0)),
            pl.BlockSpec((1, NSFF), lambda t: (0, 0)),
            pl.BlockSpec((D, NSFF), lambda t: (0, 0)),
            pl.BlockSpec((1, D), lambda t: (0, 0)),
        ],
        out_specs=(pl.BlockSpec((TBS, D), lambda t: (t, 0)),
                   pl.BlockSpec((TBS, TOPK), lambda t: (t, 0)),
                   pl.BlockSpec((TBS, TOPK), lambda t: (t, 0)),
                   pl.BlockSpec((1, E, npw), lambda t: (t, 0, 0))),
        out_shape=(jax.ShapeDtypeStruct((T, D), jnp.float32),
                   jax.ShapeDtypeStruct((T, TOPK), jnp.int32),
                   jax.ShapeDtypeStruct((T, TOPK), jnp.float32),
                   jax.ShapeDtypeStruct((T // TBS, E, npw), jnp.int32)),
    )(x2, xb16, gate_w, gate_b2, Ws1b, bs1r, Ws3b, bs3r, Ws2b, bs2r)


def _dispatch_body(eflat, cnt8, x2,
                   xs, pair_slot, block_expert,
                   eids_v, cntv, slots_v,
                   gidx, sidx, gbuf0, gbuf1, gbuf2, bev,
                   semt,
                   gsem0, gsem1, gsem2, ssem0, ssem1, ssem2):
    c = lax.axis_index("c")
    s = lax.axis_index("s")
    wid = s * NC + c

    pltpu.sync_copy(eflat.at[pl.ds(wid * CPW, CPW)], eids_v)
    pltpu.sync_copy(cnt8, cntv)

    zeros16 = jnp.zeros((16,), jnp.int32)
    lane16 = lax.broadcasted_iota(jnp.int32, (16,), 0)

    for j in range(VPW):
        toks = lax.shift_right_logical(
            lane16 + wid * CPW + j * 16, 1).astype(jnp.int32)
        gidx[j // 2, pl.ds((j % 2) * 16, 16)] = toks

    nch = CPW // GCH
    bufs = [gbuf0, gbuf1, gbuf2]
    gsems = [gsem0, gsem1, gsem2]
    ssems = [ssem0, ssem1, ssem2]
    gcopies = [None] * nch
    scopies = [None] * nch
    for ci in range(min(3, nch)):
        gcopies[ci] = pltpu.make_async_copy(
            x2.at[gidx.at[ci]], bufs[ci % 3], gsems[ci % 3])
        gcopies[ci].start()

    tot = []
    pre = []
    for e in range(E):
        v0 = cntv[e, pl.ds(0, 16)]
        v1 = cntv[e, pl.ds(16, 16)]
        tot.append(jnp.sum(v0) + jnp.sum(v1))
        pre.append(jnp.sum(jnp.where(lane16 < wid, v0, zeros16))
                   + jnp.sum(jnp.where(lane16 + 16 < wid, v1, zeros16)))
    cap = [((tot[e] + (BLK - 1)) // BLK) * BLK for e in range(E)]
    base = [jnp.int32(0)]
    for e in range(E):
        base.append(base[e] + cap[e])

    carry = [jnp.int32(0)] * E
    for j in range(VPW):
        v = eids_v[pl.ds(j * 16, 16)]
        slots = zeros16
        for e in range(E):
            ind = jnp.where(v == e, 1, 0)
            r = plsc.cumsum(ind)
            slots = slots + ind * (base[e] + pre[e] + carry[e] - 1 + r)
            carry[e] = carry[e] + r[15]
        slots_v[pl.ds(j * 16, 16)] = slots
        sidx[j // 2, pl.ds((j % 2) * 16, 16)] = slots

    ct = pltpu.make_async_copy(sidx, pair_slot.at[wid], semt)
    ct.start()

    for ci in range(nch):
        b = ci % 3
        gcopies[ci].wait()
        if ci >= 3:
            pass
        scopies[ci] = pltpu.make_async_copy(
            bufs[b], xs.at[sidx.at[ci]], ssems[b])
        scopies[ci].start()
        if ci + 3 < nch:
            scopies[ci].wait()
            gcopies[ci + 3] = pltpu.make_async_copy(
                x2.at[gidx.at[ci + 3]], bufs[b], gsems[b])
            gcopies[ci + 3].start()
    for ci in range(max(0, nch - 3), nch):
        scopies[ci].wait()

    @pl.when(wid == 0)
    def _block_expert():
        for t in range(NBLKP // 16):
            bid = lane16 + t * 16
            acc = jnp.zeros((16,), jnp.int32)
            for e in range(E):
                lo = base[e] // BLK
                hi = (base[e] + cap[e]) // BLK
                acc = acc + e * jnp.where((bid >= lo) & (bid < hi), 1, 0)
            bev[pl.ds(t * 16, 16)] = acc
        pltpu.sync_copy(bev, block_expert)

    ct.wait()


@jax.jit
def _dispatch(eflat, cnt8, x2):
    mesh = plsc.VectorSubcoreMesh(core_axis_name="c", subcore_axis_name="s",
                                  num_cores=NC, num_subcores=NSC)
    f = pl.kernel(
        _dispatch_body,
        out_type=(jax.ShapeDtypeStruct((SWLEN, D), jnp.float32),
                  jax.ShapeDtypeStruct((NW, CPW // GCH, GCH), jnp.int32),
                  jax.ShapeDtypeStruct((NBLKP,), jnp.int32)),
        mesh=mesh,
        compiler_params=pltpu.CompilerParams(needs_layout_passes=False),
        scratch_types=[
            pltpu.VMEM((CPW,), jnp.int32),
            pltpu.VMEM((E, NW), jnp.int32),
            pltpu.VMEM((CPW,), jnp.int32),
            pltpu.VMEM((CPW // GCH, GCH), jnp.int32),
            pltpu.VMEM((CPW // GCH, GCH), jnp.int32),
            pltpu.VMEM((GCH, D), jnp.float32),
            pltpu.VMEM((GCH, D), jnp.float32),
            pltpu.VMEM((GCH, D), jnp.float32),
            pltpu.VMEM((NBLKP,), jnp.int32),
            pltpu.SemaphoreType.DMA,
            pltpu.SemaphoreType.DMA,
            pltpu.SemaphoreType.DMA,
            pltpu.SemaphoreType.DMA,
            pltpu.SemaphoreType.DMA,
            pltpu.SemaphoreType.DMA,
            pltpu.SemaphoreType.DMA,
        ],
    )
    return f(eflat, cnt8, x2)


def _gffn_body(be_ref, xs_ref, w1_ref, b1_ref, w3_ref, b3_ref,
               w2_ref, b2_ref, ys_ref):
    xb = xs_ref[...].astype(jnp.bfloat16)
    h1 = lax.dot_general(xb, w1_ref[0], (((1,), (1,)), ((), ())),
                         preferred_element_type=jnp.float32) + b1_ref[0]
    h3 = lax.dot_general(xb, w3_ref[0], (((1,), (1,)), ((), ())),
                         preferred_element_type=jnp.float32) + b3_ref[0]
    h = ((h1 * jax.nn.sigmoid(h1)) * h3).astype(jnp.bfloat16)
    ye = lax.dot_general(h, w2_ref[0], (((1,), (1,)), ((), ())),
                         preferred_element_type=jnp.float32) + b2_ref[0]
    ys_ref[...] = jnp.where(jnp.abs(ye) < jnp.inf, ye, 0.0)


@jax.jit
def _gffn(block_expert, xs, W1b, b1r, W3b, b3r, W2b, b2r):
    grid_spec = pltpu.PrefetchScalarGridSpec(
        num_scalar_prefetch=1,
        grid=(NBLK,),
        in_specs=[
            pl.BlockSpec((BLK, D), lambda b, be: (b, 0)),
            pl.BlockSpec((1, FF, D), lambda b, be: (be[b], 0, 0)),
            pl.BlockSpec((1, 1, FF), lambda b, be: (be[b], 0, 0)),
            pl.BlockSpec((1, FF, D), lambda b, be: (be[b], 0, 0)),
            pl.BlockSpec((1, 1, FF), lambda b, be: (be[b], 0, 0)),
            pl.BlockSpec((1, D, FF), lambda b, be: (be[b], 0, 0)),
            pl.BlockSpec((1, 1, D), lambda b, be: (be[b], 0, 0)),
        ],
        out_specs=pl.BlockSpec((BLK, D), lambda b, be: (b, 0)),
    )
    return pl.pallas_call(
        _gffn_body,
        grid_spec=grid_spec,
        out_shape=jax.ShapeDtypeStruct((NSLOT, D), jnp.float32),
    )(block_expert, xs, W1b, b1r, W3b, b3r, W2b, b2r)


TPW = T // NW


def _combine_body(ys, pair_slot, wflat, z, y2,
                  psl, wv, ybuf0, ybuf1, zbuf0, zbuf1, outbuf0,
                  gsem0, gsem1, zsem0, zsem1, osem0, osem1):
    c = lax.axis_index("c")
    s = lax.axis_index("s")
    wid = s * NC + c

    pltpu.sync_copy(pair_slot.at[wid], psl)
    pltpu.sync_copy(wflat.at[pl.ds(wid * CPW, CPW)], wv)

    nch = CPW // GCH
    ybufs = [ybuf0, ybuf1]
    zbufs = [zbuf0, zbuf1]
    obufs = [outbuf0, outbuf0]
    gsems = [gsem0, gsem1]
    zsems = [zsem0, zsem1]
    osems = [osem0, osem1]
    t_of = lambda ci: wid * TPW + ci * (GCH // 2)
    gcopies = [None] * nch
    zcopies = [None] * nch
    ocopies = [None] * nch
    gcopies[0] = pltpu.make_async_copy(ys.at[psl.at[0]], ybufs[0], gsems[0])
    gcopies[0].start()
    zcopies[0] = pltpu.make_async_copy(z.at[pl.ds(t_of(0), GCH // 2)],
                                       zbufs[0], zsems[0])
    zcopies[0].start()
    for ci in range(nch):
        b = ci % 2
        if ci + 1 < nch:
            gcopies[ci + 1] = pltpu.make_async_copy(
                ys.at[psl.at[ci + 1]], ybufs[1 - b], gsems[1 - b])
            gcopies[ci + 1].start()
            zcopies[ci + 1] = pltpu.make_async_copy(
                z.at[pl.ds(t_of(ci + 1), GCH // 2)], zbufs[1 - b], zsems[1 - b])
            zcopies[ci + 1].start()
        gcopies[ci].wait()
        zcopies[ci].wait()
        if ci >= 1:
            ocopies[ci - 1].wait()
        ybuf = ybufs[b]
        zbuf = zbufs[b]
        outbuf = obufs[b]

        def addrow(j, _):
            pe = ci * GCH + 2 * j
            we = plsc.load_gather(wv, [jnp.full((16,), pe, jnp.int32)])
            wo = plsc.load_gather(wv, [jnp.full((16,), pe + 1, jnp.int32)])
            for l in range(D // 16):
                sl = pl.ds(l * 16, 16)
                outbuf[j, sl] = (we * ybuf[2 * j, sl]
                                 + wo * ybuf[2 * j + 1, sl] + zbuf[j, sl])
            return 0
        lax.fori_loop(0, GCH // 2, addrow, 0)
        ocopies[ci] = pltpu.make_async_copy(
            outbuf, y2.at[pl.ds(t_of(ci), GCH // 2)], osems[b])
        ocopies[ci].start()
    ocopies[nch - 1].wait()


@jax.jit
def _combine(ys, pair_slot, wflat, z):
    mesh = plsc.VectorSubcoreMesh(core_axis_name="c", subcore_axis_name="s",
                                  num_cores=NC, num_subcores=NSC)
    f = pl.kernel(
        _combine_body,
        out_type=jax.ShapeDtypeStruct((T, D), jnp.float32),
        mesh=mesh,
        compiler_params=pltpu.CompilerParams(needs_layout_passes=False),
        scratch_types=[
            pltpu.VMEM((CPW // GCH, GCH), jnp.int32),
            pltpu.VMEM((CPW,), jnp.float32),
            pltpu.VMEM((GCH, D), jnp.float32),
            pltpu.VMEM((GCH, D), jnp.float32),
            pltpu.VMEM((GCH // 2, D), jnp.float32),
            pltpu.VMEM((GCH // 2, D), jnp.float32),
            pltpu.VMEM((GCH // 2, D), jnp.float32),
            pltpu.SemaphoreType.DMA,
            pltpu.SemaphoreType.DMA,
            pltpu.SemaphoreType.DMA,
            pltpu.SemaphoreType.DMA,
            pltpu.SemaphoreType.DMA,
            pltpu.SemaphoreType.DMA,
        ],
    )
    return f(ys, pair_slot, wflat, z)


def kernel(x, gate_w, gate_b, W1, b1, W2, b2, W3, b3,
           Ws1, bs1, Ws2, bs2, Ws3, bs3):
    shape = x.shape
    x2 = x.reshape(T, D)
    xb16 = x2.astype(jnp.bfloat16)

    z, eids, wts, cnt8 = _shared(
        x2, xb16, gate_w, gate_b.reshape(1, E),
        Ws1.astype(jnp.bfloat16), bs1.reshape(1, NSFF),
        Ws3.astype(jnp.bfloat16), bs3.reshape(1, NSFF),
        Ws2.astype(jnp.bfloat16), bs2.reshape(1, D))
    cnt8f = jnp.concatenate([cnt8[i] for i in range(cnt8.shape[0])], axis=1)
    xs, pair_slot, block_expert = _dispatch(eids.reshape(P), cnt8f, x2)
    ys = _gffn(block_expert, xs,
               W1.astype(jnp.bfloat16), b1.reshape(E, 1, FF),
               W3.astype(jnp.bfloat16), b3.reshape(E, 1, FF),
               W2.astype(jnp.bfloat16), b2.reshape(E, 1, D))
    y2 = _combine(ys, pair_slot, wts.reshape(P), z)
    return y2.reshape(shape)
